# baseline (speedup 1.0000x reference)
import jax
import jax.numpy as jnp
from jax import lax
from jax.experimental import pallas as pl
from jax.experimental.pallas import tpu as pltpu
from jax.experimental.pallas import tpu_sc as plsc

NUM_ITEMS = 1000000
D = 16
TOTAL_TOK = 409600
B = 16384

NC = 2
NS = 16
SEG_HALF = B // NC
NQ = 10
ACC_SEG = SEG_HALF * NQ
TRASH = ACC_SEG
ACC_ROWS = ACC_SEG + 128
ZROWS = ACC_ROWS // NS
CHUNK = 1280
G = CHUNK // 128
SEG_PER_TILE = SEG_HALF // NS
SUB = SEG_PER_TILE // 128


def _sc_body(fid, rat, seg, splits, utab, rtab, out,
             fid2d, seg2d, cidx2d, rat_v,
             urows, orows, rtab_v,
             split_v, acc, sem_in, sem_g):
    c = lax.axis_index("c")
    s = lax.axis_index("s")

    pltpu.sync_copy(splits, split_v)
    split = jnp.max(split_v[...], axis=0)

    @pl.loop(0, CHUNK, unroll=4)
    def _z(i):
        urows[i] = jnp.zeros((D,), jnp.float32)

    zbase = s * ZROWS
    for t in range(4):
        pltpu.sync_copy(urows, acc.at[pl.ds(zbase + t * CHUNK, CHUNK)])
    pltpu.sync_copy(urows.at[pl.ds(0, ZROWS - 4 * CHUNK)],
                    acc.at[pl.ds(zbase + 4 * CHUNK, ZROWS - 4 * CHUNK)])
    plsc.subcore_barrier()

    start_c = jnp.where(c == 0, 0, split)
    end_c = jnp.where(c == 0, split, TOTAL_TOK)
    n = end_c - start_c
    m = (n + NS - 1) // NS
    w_start = start_c + s * m
    w_end = jnp.minimum(w_start + m, end_c)
    seg_base = c * SEG_HALF

    first_g = w_start // CHUNK
    last_g1 = jnp.where(w_end > w_start, (w_end - 1) // CHUNK + 1, first_g)

    lane = lax.iota(jnp.int32, 16)

    @pl.loop(first_g, last_g1)
    def _chunk(g):
        base = g * CHUNK
        descs = []
        for j in range(G):
            descs.append(pltpu.async_copy(
                fid.at[pl.ds(base + j * 128, 128)], fid2d.at[j], sem_in))
            descs.append(pltpu.async_copy(
                seg.at[pl.ds(base + j * 128, 128)], seg2d.at[j], sem_in))
        descs.append(pltpu.async_copy(rat.at[pl.ds(base, CHUNK)], rat_v, sem_in))
        for d in descs:
            d.wait()

        for r in range(G):
            @pl.loop(0, 8, unroll=4)
            def _q(k, r=r):
                off = r * 128 + k * 16
                pos = base + off + lane
                sv = seg2d[r, pl.ds(k * 16, 16)]
                rv = rat_v[pl.ds(off, 16)]
                q = ((rv - 0.5) * 2.0).astype(jnp.int32)
                cidx = (sv - seg_base) * NQ + q
                valid = (pos >= w_start) & (pos < w_end)
                cidx2d[r, pl.ds(k * 16, 16)] = jnp.where(valid, cidx, TRASH)

        gds = [pltpu.async_copy(
            utab.at[fid2d.at[j]], urows.at[pl.ds(j * 128, 128)], sem_g)
            for j in range(G)]
        for d in gds:
            d.wait()
        for j in range(G):
            pltpu.sync_copy(urows.at[pl.ds(j * 128, 128)],
                            acc.at[cidx2d.at[j]], add=True)

    plsc.subcore_barrier()

    pltpu.sync_copy(rtab, rtab_v)
    rtv = [rtab_v[q] for q in range(NQ)]

    for sb in range(SUB):
        srow0 = s * SEG_PER_TILE + sb * 128
        pltpu.sync_copy(acc.at[pl.ds(srow0 * NQ, 128 * NQ)], urows)

        @pl.loop(0, 128, unroll=2)
        def _comb(i):
            abase = i * NQ
            v = urows[abase] * rtv[0]
            for q in range(1, NQ):
                v = v + urows[abase + q] * rtv[q]
            orows[i] = v

        pltpu.sync_copy(orows, out.at[pl.ds(seg_base + srow0, 128)])


_sc_forward = pl.kernel(
    _sc_body,
    out_type=jax.ShapeDtypeStruct((B, D), jnp.float32),
    mesh=plsc.VectorSubcoreMesh(core_axis_name="c", subcore_axis_name="s"),
    scratch_types=[
        pltpu.VMEM((G, 128), jnp.int32),
        pltpu.VMEM((G, 128), jnp.int32),
        pltpu.VMEM((G, 128), jnp.int32),
        pltpu.VMEM((CHUNK,), jnp.float32),
        pltpu.VMEM((CHUNK, D), jnp.float32),
        pltpu.VMEM((128, D), jnp.float32),
        pltpu.VMEM((NQ, D), jnp.float32),
        pltpu.VMEM((16,), jnp.int32),
        pltpu.VMEM_SHARED((ACC_ROWS, D), jnp.float32),
        pltpu.SemaphoreType.DMA,
        pltpu.SemaphoreType.DMA,
    ],
    compiler_params=pltpu.CompilerParams(use_tc_tiling_on_sc=False,
                                         needs_layout_passes=False),
)


def kernel(feature_ids, feature_ratings, segment_ids, item_ids,
           user_table, rating_table, item_table, bias):
    fid = feature_ids.astype(jnp.int32)
    seg = segment_ids.astype(jnp.int32)
    iid = item_ids.astype(jnp.int32)
    split = jnp.sum((seg < SEG_HALF).astype(jnp.int32)).astype(jnp.int32)
    splits = jnp.full((16,), split, dtype=jnp.int32)
    user_sum = _sc_forward(fid, feature_ratings, seg, splits,
                           user_table, rating_table)
    return user_sum + jnp.take(item_table, iid, axis=0) + bias


# speedup vs baseline: 1.0195x; 1.0195x over previous
import jax
import jax.numpy as jnp
from jax import lax
from jax.experimental import pallas as pl
from jax.experimental.pallas import tpu as pltpu
from jax.experimental.pallas import tpu_sc as plsc

NUM_ITEMS = 1000000
D = 16
TOTAL_TOK = 409600
B = 16384

NC = 2
NS = 16
SEG_HALF = B // NC
NQ = 10
ACC_SEG = SEG_HALF * NQ
TRASH = ACC_SEG
ACC_ROWS = ACC_SEG + 128
ZROWS = ACC_ROWS // NS
CHUNK = 1280
G = CHUNK // 128
SEG_PER_TILE = SEG_HALF // NS
SUB = SEG_PER_TILE // 128


def _sc_body(fid, rat, seg, splits, utab, rtab, out,
             fid2d, seg2d, cidx2d, rat_v,
             urows, orows, rtab_v,
             split_v, acc, sem_in, sem_g, sem_s):
    c = lax.axis_index("c")
    s = lax.axis_index("s")

    pltpu.sync_copy(splits, split_v)
    split = jnp.max(split_v[...], axis=0)

    @pl.loop(0, CHUNK, unroll=4)
    def _z(i):
        urows[i] = jnp.zeros((D,), jnp.float32)

    zbase = s * ZROWS
    for t in range(4):
        pltpu.sync_copy(urows, acc.at[pl.ds(zbase + t * CHUNK, CHUNK)])
    pltpu.sync_copy(urows.at[pl.ds(0, ZROWS - 4 * CHUNK)],
                    acc.at[pl.ds(zbase + 4 * CHUNK, ZROWS - 4 * CHUNK)])
    plsc.subcore_barrier()

    start_c = jnp.where(c == 0, 0, split)
    end_c = jnp.where(c == 0, split, TOTAL_TOK)
    n = end_c - start_c
    m = (n + NS - 1) // NS
    w_start = start_c + s * m
    w_end = jnp.minimum(w_start + m, end_c)
    seg_base = c * SEG_HALF

    first_g = w_start // CHUNK
    last_g1 = jnp.where(w_end > w_start, (w_end - 1) // CHUNK + 1, first_g)

    lane = lax.iota(jnp.int32, 16)

    def issue_inputs(g):
        base = jnp.minimum(g * CHUNK, TOTAL_TOK - CHUNK)
        for j in range(G):
            pltpu.async_copy(fid.at[pl.ds(base + j * 128, 128)],
                             fid2d.at[j], sem_in)
            pltpu.async_copy(seg.at[pl.ds(base + j * 128, 128)],
                             seg2d.at[j], sem_in)
        pltpu.async_copy(rat.at[pl.ds(base, CHUNK)], rat_v, sem_in)

    def wait_inputs():
        for j in range(G):
            pltpu.make_async_copy(fid.at[pl.ds(0, 128)], fid2d.at[j],
                                  sem_in).wait()
            pltpu.make_async_copy(seg.at[pl.ds(0, 128)], seg2d.at[j],
                                  sem_in).wait()
        pltpu.make_async_copy(rat.at[pl.ds(0, CHUNK)], rat_v, sem_in).wait()

    issue_inputs(first_g)

    @pl.loop(first_g, last_g1)
    def _chunk(g):
        base = g * CHUNK
        wait_inputs()

        for r in range(G):
            @pl.loop(0, 8, unroll=4)
            def _q(k, r=r):
                off = r * 128 + k * 16
                pos = base + off + lane
                sv = seg2d[r, pl.ds(k * 16, 16)]
                rv = rat_v[pl.ds(off, 16)]
                q = ((rv - 0.5) * 2.0).astype(jnp.int32)
                cidx = (sv - seg_base) * NQ + q
                valid = (pos >= w_start) & (pos < w_end)
                cidx2d[r, pl.ds(k * 16, 16)] = jnp.where(valid, cidx, TRASH)

        gds = [pltpu.async_copy(
            utab.at[fid2d.at[j]], urows.at[pl.ds(j * 128, 128)], sem_g)
            for j in range(G)]
        for d in gds:
            d.wait()
        issue_inputs(g + 1)
        sds = [pltpu.async_copy(
            urows.at[pl.ds(j * 128, 128)], acc.at[cidx2d.at[j]],
            sem_s, add=True) for j in range(G)]
        for d in sds:
            d.wait()

    wait_inputs()
    plsc.subcore_barrier()

    pltpu.sync_copy(rtab, rtab_v)
    rtv = [rtab_v[q] for q in range(NQ)]

    for sb in range(SUB):
        srow0 = s * SEG_PER_TILE + sb * 128
        pltpu.sync_copy(acc.at[pl.ds(srow0 * NQ, 128 * NQ)], urows)

        @pl.loop(0, 128, unroll=2)
        def _comb(i):
            abase = i * NQ
            v = urows[abase] * rtv[0]
            for q in range(1, NQ):
                v = v + urows[abase + q] * rtv[q]
            orows[i] = v

        pltpu.sync_copy(orows, out.at[pl.ds(seg_base + srow0, 128)])


_sc_forward = pl.kernel(
    _sc_body,
    out_type=jax.ShapeDtypeStruct((B, D), jnp.float32),
    mesh=plsc.VectorSubcoreMesh(core_axis_name="c", subcore_axis_name="s"),
    scratch_types=[
        pltpu.VMEM((G, 128), jnp.int32),
        pltpu.VMEM((G, 128), jnp.int32),
        pltpu.VMEM((G, 128), jnp.int32),
        pltpu.VMEM((CHUNK,), jnp.float32),
        pltpu.VMEM((CHUNK, D), jnp.float32),
        pltpu.VMEM((128, D), jnp.float32),
        pltpu.VMEM((NQ, D), jnp.float32),
        pltpu.VMEM((16,), jnp.int32),
        pltpu.VMEM_SHARED((ACC_ROWS, D), jnp.float32),
        pltpu.SemaphoreType.DMA,
        pltpu.SemaphoreType.DMA,
        pltpu.SemaphoreType.DMA,
    ],
    compiler_params=pltpu.CompilerParams(use_tc_tiling_on_sc=False,
                                         needs_layout_passes=False),
)


def kernel(feature_ids, feature_ratings, segment_ids, item_ids,
           user_table, rating_table, item_table, bias):
    fid = feature_ids.astype(jnp.int32)
    seg = segment_ids.astype(jnp.int32)
    iid = item_ids.astype(jnp.int32)
    split = jnp.sum((seg < SEG_HALF).astype(jnp.int32)).astype(jnp.int32)
    splits = jnp.full((16,), split, dtype=jnp.int32)
    user_sum = _sc_forward(fid, feature_ratings, seg, splits,
                           user_table, rating_table)
    return user_sum + jnp.take(item_table, iid, axis=0) + bias
